# baseline (speedup 1.0000x reference)
import functools

import jax
import jax.numpy as jnp
from jax import lax
from jax.experimental import pallas as pl
from jax.experimental.pallas import tpu as pltpu

_B, _T, _N = 16, 2048, 32
_K, _S = 128, 16
_P = _K * _B
_NBLK = 2
_P2 = _P // _NBLK
_LOGC = 4.0


def _crf_body(lpsp_ref, tpv_ref, lp0_ref, tgt0_ref, ubase_ref, lenp_ref,
              tr_ref, st_ref, en_ref, out_ref, m_ref):
    f32 = jnp.float32
    bf16 = jnp.bfloat16
    E = jnp.exp(tr_ref[...])
    Tbf = tr_ref[...].astype(bf16)
    en = en_ref[...]

    Et = jnp.concatenate([E.astype(bf16)] * 4, axis=0)
    Et = jnp.concatenate([Et] * 4, axis=1)
    ri = lax.broadcasted_iota(jnp.int32, (128, 128), 0)
    li = lax.broadcasted_iota(jnp.int32, (128, 128), 1)
    Ebig = jnp.where((ri // _N) == (li // _N), Et, jnp.zeros((), bf16))

    ihi3 = lax.broadcasted_iota(jnp.int32, (_P2, 8, 128), 1)
    ln3 = lax.broadcasted_iota(jnp.int32, (_P2, 8, 128), 2)
    eyep = ((ihi3 * 4 + ln3 // _N) == (ln3 % _N)).astype(bf16)

    iota2bf = lax.broadcasted_iota(jnp.int32, (_P2, _N), 1).astype(bf16)

    lp0 = lp0_ref[...]
    iota0 = lax.broadcasted_iota(jnp.int32, (_B, _N), 1)
    oh0 = (tgt0_ref[...] == iota0).astype(f32)
    start_sel = jnp.sum(st_ref[...] * oh0, axis=1, keepdims=True)
    em0 = jnp.sum(lp0 * oh0, axis=1, keepdims=True)

    def block(blk, acc):
        em_sum, tr_sum, end_sel = acc
        r0 = blk * _P2
        ub128 = ubase_ref[pl.ds(r0, _P2), :]
        len128 = lenp_ref[pl.ds(r0, _P2), :]

        def step_parts(s, tc_s, carry):
            Mbf, em_acc, tr_acc, end_acc = carry
            lp128 = lpsp_ref[s, pl.ds(r0, _P2), :]
            tp_s = tpv_ref[s, pl.ds(r0, _P2), :]
            ohp = (tp_s == iota2bf).astype(bf16)
            ohc = (tc_s == iota2bf).astype(bf16)
            vs128 = (ub128 + (s + 1)) < len128
            vs32 = vs128[:, :_N]
            valf = vs32.astype(f32)
            lastf = ((ub128[:, :_N] + s) == (len128[:, :_N] - 1)).astype(f32)

            lp_s = lp128[:, :_N].astype(f32)
            em_acc = em_acc + lp_s * ohc.astype(f32) * valf
            rows = lax.dot(ohp, Tbf, preferred_element_type=f32)
            tr_acc = tr_acc + rows * ohc.astype(f32) * valf
            end_acc = end_acc + ohp.astype(f32) * lastf

            dp = jnp.exp(lp128.astype(f32) - _LOGC)
            R = lax.dot(Mbf.reshape(_P2 * 8, 128), Ebig,
                        preferred_element_type=f32).reshape(_P2, 8, 128)
            R = R * dp[:, None, :]
            Mbf = jnp.where(vs128[:, None, :], R.astype(bf16), Mbf)
            return Mbf, em_acc, tr_acc, end_acc

        def step(s, carry):
            tc_s = tpv_ref[s + 1, pl.ds(r0, _P2), :]
            return step_parts(s, tc_s, carry)

        z2 = jnp.zeros((_P2, _N), f32)
        carry = lax.fori_loop(0, _S - 1, step, (eyep, z2, z2, z2))
        tc_last = tpv_ref[0, pl.ds(r0 + _B, _P2), :]
        Mbf, em_acc, tr_acc, end_acc = step_parts(_S - 1, tc_last, carry)

        m_ref[pl.ds(r0 * 8, _P2 * 8)] = Mbf.reshape(_P2 * 8, 128)

        def _per_b(x2):
            xp = jnp.sum(x2, axis=1).reshape(_P2 // _B, _B)
            return jnp.sum(xp, axis=0)[:, None]

        em_sum = em_sum + _per_b(em_acc)
        tr_sum = tr_sum + _per_b(tr_acc)
        end_sel = end_sel + _per_b(end_acc * en)
        return em_sum, tr_sum, end_sel

    zero = jnp.zeros((_B, 1), f32)
    em_sum, tr_sum, end_sel = lax.fori_loop(
        0, _NBLK, block, (zero, zero, zero))

    lens = lenp_ref[pl.ds(0, _B), :][:, :1]
    alpha0 = st_ref[...] + lp0

    r8 = lax.broadcasted_iota(jnp.int32, (128, 256), 0)
    l8 = lax.broadcasted_iota(jnp.int32, (128, 256), 1)
    bmask = ((r8 % 8) == (l8 // _N)).astype(bf16)
    rq = lax.broadcasted_iota(jnp.int32, (256, 128), 0)
    lq = lax.broadcasted_iota(jnp.int32, (256, 128), 1)
    Q = ((rq % _N) == ((rq // _N) * 4 + lq // _N)).astype(bf16)
    rs = lax.broadcasted_iota(jnp.int32, (128, _N), 0)
    ls = lax.broadcasted_iota(jnp.int32, (128, _N), 1)
    SS = ((rs % _N) == ls).astype(f32)

    def fold(c, alpha):
        Mcp = m_ref[pl.ds(c * 128, 128)].astype(f32)
        nval = jnp.clip(lens - 1 - c * _S, 0, _S).astype(f32)
        m = jnp.max(alpha, axis=1, keepdims=True)
        e = (jnp.exp(alpha - m)).astype(bf16)
        et = jnp.concatenate([e] * 8, axis=1)
        ebd = jnp.broadcast_to(et[:, None, :],
                               (_B, 8, 256)).reshape(128, 256) * bmask
        eexp = lax.dot(ebd, Q, preferred_element_type=f32)
        X = jnp.sum((eexp * Mcp).reshape(_B, 8, 128), axis=1)
        prod = lax.dot(X, SS, preferred_element_type=f32)
        return jnp.log(prod) + m + nval * _LOGC

    alpha = lax.fori_loop(0, _K, fold, alpha0)

    z = alpha + en
    m2 = jnp.max(z, axis=1, keepdims=True)
    logZ = jnp.log(jnp.sum(jnp.exp(z - m2), axis=1, keepdims=True)) + m2

    out_ref[...] = start_sel + em0 + em_sum + tr_sum + end_sel - logZ


@functools.partial(jax.jit, static_argnames=())
def kernel(log_potentials, target, lengths, transition, start_transition,
           end_transition):
    lp_steps = jnp.concatenate(
        [log_potentials[:, 1:, :], log_potentials[:, :1, :]], axis=1)
    lp4 = lp_steps.reshape(_B, _K, _S, _N).astype(jnp.bfloat16)
    lpsp = jnp.tile(jnp.transpose(lp4, (2, 1, 0, 3)).reshape(_S, _P, _N),
                    (1, 1, 4))
    tg3 = jnp.transpose(target.reshape(_B, _K, _S), (2, 1, 0))
    tpv = jnp.broadcast_to(
        tg3.reshape(_S, _P, 1).astype(jnp.bfloat16), (_S, _P, _N))
    tpv = jnp.pad(tpv, ((0, 0), (0, _B), (0, 0)))
    ubase = jnp.broadcast_to(
        ((jnp.arange(_P, dtype=jnp.int32) // _B) * _S)[:, None], (_P, 128))
    lenp = jnp.broadcast_to(
        jnp.tile(lengths.astype(jnp.int32), _K)[:, None], (_P, 128))
    out = pl.pallas_call(
        _crf_body,
        out_shape=jax.ShapeDtypeStruct((_B, 1), jnp.float32),
        scratch_shapes=[pltpu.VMEM((_P * 8, 128), jnp.bfloat16)],
    )(lpsp, tpv, log_potentials[:, 0, :], target[:, :1], ubase, lenp,
      transition, start_transition[None, :], end_transition[None, :])
    return out[:, 0]


# speedup vs baseline: 1.0522x; 1.0522x over previous
import functools

import jax
import jax.numpy as jnp
from jax import lax
from jax.experimental import pallas as pl
from jax.experimental.pallas import tpu as pltpu

_B, _T, _N = 16, 2048, 32
_K, _S = 128, 16
_P = _K * _B
_NBLK = 2
_P2 = _P // _NBLK
_LOGC = 4.0


def _crf_body(lpsp_ref, tpv_ref, lp0_ref, tgt0_ref, ubase_ref, lenp_ref,
              tr_ref, st_ref, en_ref, out_ref, m_ref):
    f32 = jnp.float32
    bf16 = jnp.bfloat16
    E = jnp.exp(tr_ref[...])
    Tbf = tr_ref[...].astype(bf16)
    en = en_ref[...]

    Et = jnp.concatenate([E.astype(bf16)] * 4, axis=0)
    Et = jnp.concatenate([Et] * 4, axis=1)
    ri = lax.broadcasted_iota(jnp.int32, (128, 128), 0)
    li = lax.broadcasted_iota(jnp.int32, (128, 128), 1)
    Ebig = jnp.where((ri // _N) == (li // _N), Et, jnp.zeros((), bf16))

    ihi3 = lax.broadcasted_iota(jnp.int32, (_P2, 8, 128), 1)
    ln3 = lax.broadcasted_iota(jnp.int32, (_P2, 8, 128), 2)
    eyep = ((ihi3 * 4 + ln3 // _N) == (ln3 % _N)).astype(bf16)

    iota2bf = lax.broadcasted_iota(jnp.int32, (_P2, _N), 1).astype(bf16)

    lp0 = lp0_ref[...]
    iota0 = lax.broadcasted_iota(jnp.int32, (_B, _N), 1)
    oh0 = (tgt0_ref[...] == iota0).astype(f32)
    start_sel = jnp.sum(st_ref[...] * oh0, axis=1, keepdims=True)
    em0 = jnp.sum(lp0 * oh0, axis=1, keepdims=True)

    def block(blk, acc):
        em_sum, tr_sum, end_sel = acc
        r0 = blk * _P2
        ub128 = ubase_ref[pl.ds(r0, _P2), :]
        len128 = lenp_ref[pl.ds(r0, _P2), :]

        def step_parts(s, tc_s, carry):
            Mbf, em_acc, tr_acc, end_acc = carry
            lp128 = lpsp_ref[s, pl.ds(r0, _P2), :]
            tp_s = tpv_ref[s, pl.ds(r0, _P2), :]
            ohp = (tp_s == iota2bf).astype(bf16)
            ohc = (tc_s == iota2bf).astype(bf16)
            vs128 = (ub128 + (s + 1)) < len128
            vs32 = vs128[:, :_N]
            valf = vs32.astype(f32)
            lastf = ((ub128[:, :_N] + s) == (len128[:, :_N] - 1)).astype(f32)

            lp_s = lp128[:, :_N].astype(f32)
            em_acc = em_acc + lp_s * ohc.astype(f32) * valf
            rows = lax.dot(ohp, Tbf, preferred_element_type=f32)
            tr_acc = tr_acc + rows * ohc.astype(f32) * valf
            end_acc = end_acc + ohp.astype(f32) * lastf

            dp = jnp.exp(lp128.astype(f32) - _LOGC)
            R = lax.dot(Mbf.reshape(_P2 * 8, 128), Ebig,
                        preferred_element_type=f32).reshape(_P2, 8, 128)
            R = R * dp[:, None, :]
            Mbf = jnp.where(vs128[:, None, :], R.astype(bf16), Mbf)
            return Mbf, em_acc, tr_acc, end_acc

        def step(s, carry):
            tc_s = tpv_ref[s + 1, pl.ds(r0, _P2), :]
            return step_parts(s, tc_s, carry)

        z2 = jnp.zeros((_P2, _N), f32)
        carry = lax.fori_loop(0, _S - 1, step, (eyep, z2, z2, z2))
        tc_last = tpv_ref[0, pl.ds(r0 + _B, _P2), :]
        Mbf, em_acc, tr_acc, end_acc = step_parts(_S - 1, tc_last, carry)

        m_ref[pl.ds(r0 * 8, _P2 * 8)] = Mbf.reshape(_P2 * 8, 128)

        def _per_b(x2):
            xp = jnp.sum(x2, axis=1).reshape(_P2 // _B, _B)
            return jnp.sum(xp, axis=0)[:, None]

        em_sum = em_sum + _per_b(em_acc)
        tr_sum = tr_sum + _per_b(tr_acc)
        end_sel = end_sel + _per_b(end_acc * en)
        return em_sum, tr_sum, end_sel

    zero = jnp.zeros((_B, 1), f32)
    em_sum, tr_sum, end_sel = lax.fori_loop(
        0, _NBLK, block, (zero, zero, zero))

    lens = lenp_ref[pl.ds(0, _B), :][:, :1]
    alpha0 = st_ref[...] + lp0

    rr = lax.broadcasted_iota(jnp.int32, (128, _N), 0)
    lr = lax.broadcasted_iota(jnp.int32, (128, _N), 1)
    rmask = ((rr % 8) == (lr // 4)).astype(bf16)
    rw = lax.broadcasted_iota(jnp.int32, (_N, 128), 0)
    lw = lax.broadcasted_iota(jnp.int32, (_N, 128), 1)
    W2 = ((rw % 4) == (lw // _N)).astype(bf16)
    rs = lax.broadcasted_iota(jnp.int32, (128, _N), 0)
    ls = lax.broadcasted_iota(jnp.int32, (128, _N), 1)
    SS = ((rs % _N) == ls).astype(f32)

    def fold(c, alpha):
        Mcp = m_ref[pl.ds(c * 128, 128)].astype(f32)
        nval = jnp.clip(lens - 1 - c * _S, 0, _S).astype(f32)
        m = jnp.max(alpha, axis=1, keepdims=True)
        e = (jnp.exp(alpha - m)).astype(bf16)
        erep = jnp.broadcast_to(e[:, None, :], (_B, 8, _N)).reshape(128, _N)
        eexp = lax.dot(erep * rmask, W2, preferred_element_type=f32)
        X = jnp.sum((eexp * Mcp).reshape(_B, 8, 128), axis=1)
        prod = lax.dot(X, SS, preferred_element_type=f32)
        return jnp.log(prod) + m + nval * _LOGC

    alpha = lax.fori_loop(0, _K, fold, alpha0)

    z = alpha + en
    m2 = jnp.max(z, axis=1, keepdims=True)
    logZ = jnp.log(jnp.sum(jnp.exp(z - m2), axis=1, keepdims=True)) + m2

    out_ref[...] = start_sel + em0 + em_sum + tr_sum + end_sel - logZ


@functools.partial(jax.jit, static_argnames=())
def kernel(log_potentials, target, lengths, transition, start_transition,
           end_transition):
    lp_steps = jnp.concatenate(
        [log_potentials[:, 1:, :], log_potentials[:, :1, :]], axis=1)
    lp4 = lp_steps.reshape(_B, _K, _S, _N).astype(jnp.bfloat16)
    lpsp = jnp.tile(jnp.transpose(lp4, (2, 1, 0, 3)).reshape(_S, _P, _N),
                    (1, 1, 4))
    tg3 = jnp.transpose(target.reshape(_B, _K, _S), (2, 1, 0))
    tpv = jnp.broadcast_to(
        tg3.reshape(_S, _P, 1).astype(jnp.bfloat16), (_S, _P, _N))
    tpv = jnp.pad(tpv, ((0, 0), (0, _B), (0, 0)))
    ubase = jnp.broadcast_to(
        ((jnp.arange(_P, dtype=jnp.int32) // _B) * _S)[:, None], (_P, 128))
    lenp = jnp.broadcast_to(
        jnp.tile(lengths.astype(jnp.int32), _K)[:, None], (_P, 128))
    out = pl.pallas_call(
        _crf_body,
        out_shape=jax.ShapeDtypeStruct((_B, 1), jnp.float32),
        scratch_shapes=[pltpu.VMEM((_P * 8, 128), jnp.bfloat16)],
    )(lpsp, tpv, log_potentials[:, 0, :], target[:, :1], ubase, lenp,
      transition, start_transition[None, :], end_transition[None, :])
    return out[:, 0]


# speedup vs baseline: 1.1295x; 1.0735x over previous
import functools

import jax
import jax.numpy as jnp
from jax import lax
from jax.experimental import pallas as pl
from jax.experimental.pallas import tpu as pltpu

_B, _T, _N = 16, 2048, 32
_K, _S = 64, 32
_P = _K * _B
_NBLK = 2
_P2 = _P // _NBLK
_LOGC = 3.4


def _crf_body(lpsp_ref, tpv_ref, lp0_ref, tgt0_ref, ubase_ref, lenp_ref,
              tr_ref, st_ref, en_ref, out_ref, m_ref):
    f32 = jnp.float32
    bf16 = jnp.bfloat16
    E = jnp.exp(tr_ref[...])
    Tbf = tr_ref[...].astype(bf16)
    en = en_ref[...]

    Et = jnp.concatenate([E.astype(bf16)] * 4, axis=0)
    Et = jnp.concatenate([Et] * 4, axis=1)
    ri = lax.broadcasted_iota(jnp.int32, (128, 128), 0)
    li = lax.broadcasted_iota(jnp.int32, (128, 128), 1)
    Ebig = jnp.where((ri // _N) == (li // _N), Et, jnp.zeros((), bf16))

    ihi3 = lax.broadcasted_iota(jnp.int32, (_P2, 8, 128), 1)
    ln3 = lax.broadcasted_iota(jnp.int32, (_P2, 8, 128), 2)
    eyep = ((ihi3 * 4 + ln3 // _N) == (ln3 % _N)).astype(bf16)

    iota2bf = lax.broadcasted_iota(jnp.int32, (_P2, _N), 1).astype(bf16)

    lp0 = lp0_ref[...]
    iota0 = lax.broadcasted_iota(jnp.int32, (_B, _N), 1)
    oh0 = (tgt0_ref[...] == iota0).astype(f32)
    start_sel = jnp.sum(st_ref[...] * oh0, axis=1, keepdims=True)
    em0 = jnp.sum(lp0 * oh0, axis=1, keepdims=True)

    def block(blk, acc):
        em_sum, tr_sum, end_sel = acc
        r0 = blk * _P2
        ub128 = ubase_ref[pl.ds(r0, _P2), :]
        len128 = lenp_ref[pl.ds(r0, _P2), :]

        def step_parts(s, tc_s, carry):
            Mbf, em_acc, tr_acc, end_acc = carry
            lp128 = lpsp_ref[s, pl.ds(r0, _P2), :]
            tp_s = tpv_ref[s, pl.ds(r0, _P2), :]
            ohp = (tp_s == iota2bf).astype(bf16)
            ohc = (tc_s == iota2bf).astype(bf16)
            vs128 = (ub128 + (s + 1)) < len128
            vs32 = vs128[:, :_N]
            valf = vs32.astype(f32)
            lastf = ((ub128[:, :_N] + s) == (len128[:, :_N] - 1)).astype(f32)

            lp_s = lp128[:, :_N].astype(f32)
            em_acc = em_acc + lp_s * ohc.astype(f32) * valf
            rows = lax.dot(ohp, Tbf, preferred_element_type=f32)
            tr_acc = tr_acc + rows * ohc.astype(f32) * valf
            end_acc = end_acc + ohp.astype(f32) * lastf

            dp = jnp.exp(lp128.astype(f32) - _LOGC)
            R = lax.dot(Mbf.reshape(_P2 * 8, 128), Ebig,
                        preferred_element_type=f32).reshape(_P2, 8, 128)
            R = R * dp[:, None, :]
            Mbf = jnp.where(vs128[:, None, :], R.astype(bf16), Mbf)
            return Mbf, em_acc, tr_acc, end_acc

        def step(s, carry):
            tc_s = tpv_ref[s + 1, pl.ds(r0, _P2), :]
            return step_parts(s, tc_s, carry)

        z2 = jnp.zeros((_P2, _N), f32)
        carry = lax.fori_loop(0, _S - 1, step, (eyep, z2, z2, z2))
        tc_last = tpv_ref[0, pl.ds(r0 + _B, _P2), :]
        Mbf, em_acc, tr_acc, end_acc = step_parts(_S - 1, tc_last, carry)

        m_ref[pl.ds(r0 * 8, _P2 * 8)] = Mbf.reshape(_P2 * 8, 128)

        def _per_b(x2):
            xp = jnp.sum(x2, axis=1).reshape(_P2 // _B, _B)
            return jnp.sum(xp, axis=0)[:, None]

        em_sum = em_sum + _per_b(em_acc)
        tr_sum = tr_sum + _per_b(tr_acc)
        end_sel = end_sel + _per_b(end_acc * en)
        return em_sum, tr_sum, end_sel

    zero = jnp.zeros((_B, 1), f32)
    em_sum, tr_sum, end_sel = lax.fori_loop(
        0, _NBLK, block, (zero, zero, zero))

    lens = lenp_ref[pl.ds(0, _B), :][:, :1]
    alpha0 = st_ref[...] + lp0

    rr = lax.broadcasted_iota(jnp.int32, (128, _N), 0)
    lr = lax.broadcasted_iota(jnp.int32, (128, _N), 1)
    rmask = ((rr % 8) == (lr // 4)).astype(bf16)
    rw = lax.broadcasted_iota(jnp.int32, (_N, 128), 0)
    lw = lax.broadcasted_iota(jnp.int32, (_N, 128), 1)
    W2 = ((rw % 4) == (lw // _N)).astype(bf16)
    rs = lax.broadcasted_iota(jnp.int32, (128, _N), 0)
    ls = lax.broadcasted_iota(jnp.int32, (128, _N), 1)
    SS = ((rs % _N) == ls).astype(f32)

    def fold(c, alpha):
        Mcp = m_ref[pl.ds(c * 128, 128)].astype(f32)
        nval = jnp.clip(lens - 1 - c * _S, 0, _S).astype(f32)
        m = jnp.max(alpha, axis=1, keepdims=True)
        e = (jnp.exp(alpha - m)).astype(bf16)
        erep = jnp.broadcast_to(e[:, None, :], (_B, 8, _N)).reshape(128, _N)
        eexp = lax.dot(erep * rmask, W2, preferred_element_type=f32)
        X = jnp.sum((eexp * Mcp).reshape(_B, 8, 128), axis=1)
        prod = lax.dot(X, SS, preferred_element_type=f32)
        return jnp.log(prod) + m + nval * _LOGC

    alpha = lax.fori_loop(0, _K, fold, alpha0)

    z = alpha + en
    m2 = jnp.max(z, axis=1, keepdims=True)
    logZ = jnp.log(jnp.sum(jnp.exp(z - m2), axis=1, keepdims=True)) + m2

    out_ref[...] = start_sel + em0 + em_sum + tr_sum + end_sel - logZ


@functools.partial(jax.jit, static_argnames=())
def kernel(log_potentials, target, lengths, transition, start_transition,
           end_transition):
    lp_steps = jnp.concatenate(
        [log_potentials[:, 1:, :], log_potentials[:, :1, :]], axis=1)
    lp4 = lp_steps.reshape(_B, _K, _S, _N).astype(jnp.bfloat16)
    lpsp = jnp.tile(jnp.transpose(lp4, (2, 1, 0, 3)).reshape(_S, _P, _N),
                    (1, 1, 4))
    tg3 = jnp.transpose(target.reshape(_B, _K, _S), (2, 1, 0))
    tpv = jnp.broadcast_to(
        tg3.reshape(_S, _P, 1).astype(jnp.bfloat16), (_S, _P, _N))
    tpv = jnp.pad(tpv, ((0, 0), (0, _B), (0, 0)))
    ubase = jnp.broadcast_to(
        ((jnp.arange(_P, dtype=jnp.int32) // _B) * _S)[:, None], (_P, 128))
    lenp = jnp.broadcast_to(
        jnp.tile(lengths.astype(jnp.int32), _K)[:, None], (_P, 128))
    out = pl.pallas_call(
        _crf_body,
        out_shape=jax.ShapeDtypeStruct((_B, 1), jnp.float32),
        scratch_shapes=[pltpu.VMEM((_P * 8, 128), jnp.bfloat16)],
    )(lpsp, tpv, log_potentials[:, 0, :], target[:, :1], ubase, lenp,
      transition, start_transition[None, :], end_transition[None, :])
    return out[:, 0]
